# baseline (speedup 1.0000x reference)
import functools

import jax
import jax.numpy as jnp
from jax import lax
from jax.experimental import pallas as pl
from jax.experimental.pallas import tpu as pltpu
from jax.experimental.pallas import tpu_sc as plsc

B = 1024
L = 200
EMB = 64
N = B * L
NC, NS = 2, 16
NW = NC * NS
CHUNK = 128
NCH = N // CHUNK // NW
NBUF = 5
AHEAD = 3


@functools.partial(
    pl.kernel,
    out_type=jax.ShapeDtypeStruct((N, EMB), jnp.float32),
    mesh=plsc.VectorSubcoreMesh(core_axis_name="c", subcore_axis_name="s"),
    compiler_params=pltpu.CompilerParams(use_tc_tiling_on_sc=False),
    scratch_types=(
        [pltpu.VMEM((NCH, CHUNK), jnp.int32)]
        + [pltpu.VMEM((CHUNK, EMB), jnp.float32) for _ in range(NBUF)]
        + [pltpu.SemaphoreType.DMA for _ in range(2 * NBUF)]
    ),
)
def _emb_gather(idx_hbm, table_hbm, out_hbm, *refs):
    idx_v = refs[0]
    rows = refs[1:1 + NBUF]
    sem_g = refs[1 + NBUF:1 + 2 * NBUF]
    sem_w = refs[1 + 2 * NBUF:1 + 3 * NBUF]

    wid = lax.axis_index("s") * NC + lax.axis_index("c")
    pltpu.sync_copy(idx_hbm.at[wid], idx_v)
    gbase = wid * NCH

    for b in range(AHEAD):
        pltpu.async_copy(table_hbm.at[idx_v.at[b]], rows[b], sem_g[b])

    def outer(t, carry):
        for b in range(NBUF):
            c = t * NBUF + b
            pltpu.make_async_copy(
                table_hbm.at[idx_v.at[c]], rows[b], sem_g[b]
            ).wait()
            pltpu.async_copy(
                rows[b], out_hbm.at[pl.ds((gbase + c) * CHUNK, CHUNK)],
                sem_w[b],
            )

            bn = (b + AHEAD) % NBUF
            cn = c + AHEAD

            @pl.when(cn < NCH)
            def _issue(bn=bn, cn=cn):
                @pl.when(cn >= NBUF)
                def _drain():
                    pltpu.make_async_copy(
                        rows[bn], out_hbm.at[pl.ds(0, CHUNK)], sem_w[bn]
                    ).wait()

                pltpu.async_copy(
                    table_hbm.at[idx_v.at[cn]], rows[bn], sem_g[bn]
                )

        return carry

    lax.fori_loop(0, NCH // NBUF, outer, 0)
    for b in range(NBUF):
        pltpu.make_async_copy(
            rows[b], out_hbm.at[pl.ds(0, CHUNK)], sem_w[b]
        ).wait()


def kernel(x, table, pos_emb):
    idx = x.astype(jnp.int32).reshape(NW, NCH, CHUNK)
    out = _emb_gather(idx, table)
    return out.reshape(B, L, EMB) + pos_emb


# speedup vs baseline: 1.1462x; 1.1462x over previous
import functools

import jax
import jax.numpy as jnp
from jax import lax
from jax.experimental import pallas as pl
from jax.experimental.pallas import tpu as pltpu
from jax.experimental.pallas import tpu_sc as plsc

B = 1024
L = 200
EMB = 64
N = B * L
NC, NS = 2, 16
NW = NC * NS
CHUNK = 128
NCH = N // CHUNK // NW
NBUF = 5
AHEAD = 3
PADW = 128


@functools.partial(
    pl.kernel,
    out_type=jax.ShapeDtypeStruct((N, PADW), jnp.float32),
    mesh=plsc.VectorSubcoreMesh(core_axis_name="c", subcore_axis_name="s"),
    compiler_params=pltpu.CompilerParams(use_tc_tiling_on_sc=False),
    scratch_types=(
        [pltpu.VMEM((NCH, CHUNK), jnp.int32)]
        + [pltpu.VMEM((CHUNK, PADW), jnp.float32) for _ in range(NBUF)]
        + [pltpu.SemaphoreType.DMA for _ in range(2 * NBUF)]
    ),
)
def _emb_gather(idx_hbm, table_hbm, out_hbm, *refs):
    idx_v = refs[0]
    rows = refs[1:1 + NBUF]
    sem_g = refs[1 + NBUF:1 + 2 * NBUF]
    sem_w = refs[1 + 2 * NBUF:1 + 3 * NBUF]

    wid = lax.axis_index("s") * NC + lax.axis_index("c")
    pltpu.sync_copy(idx_hbm.at[wid], idx_v)
    gbase = wid * NCH

    for b in range(AHEAD):
        pltpu.async_copy(table_hbm.at[idx_v.at[b]], rows[b], sem_g[b])

    def outer(t, carry):
        for b in range(NBUF):
            c = t * NBUF + b
            pltpu.make_async_copy(
                table_hbm.at[idx_v.at[c]], rows[b], sem_g[b]
            ).wait()
            pltpu.async_copy(
                rows[b], out_hbm.at[pl.ds((gbase + c) * CHUNK, CHUNK)],
                sem_w[b],
            )

            bn = (b + AHEAD) % NBUF
            cn = c + AHEAD

            @pl.when(cn < NCH)
            def _issue(bn=bn, cn=cn):
                @pl.when(cn >= NBUF)
                def _drain():
                    pltpu.make_async_copy(
                        rows[bn], out_hbm.at[pl.ds(0, CHUNK)], sem_w[bn]
                    ).wait()

                pltpu.async_copy(
                    table_hbm.at[idx_v.at[cn]], rows[bn], sem_g[bn]
                )

        return carry

    lax.fori_loop(0, NCH // NBUF, outer, 0)
    for b in range(NBUF):
        pltpu.make_async_copy(
            rows[b], out_hbm.at[pl.ds(0, CHUNK)], sem_w[b]
        ).wait()


def kernel(x, table, pos_emb):
    idx = x.astype(jnp.int32).reshape(NW, NCH, CHUNK)
    tpad = jnp.pad(table, ((0, 0), (0, PADW - EMB)))
    out = _emb_gather(idx, tpad)
    return out[:, :EMB].reshape(B, L, EMB) + pos_emb
